# Initial kernel scaffold; baseline (speedup 1.0000x reference)
import jax
import jax.numpy as jnp
from jax.experimental import pallas as pl


def kernel(x, edge_index, W1, b1, W2, b2):
    raise NotImplementedError("write your pallas kernel here")


# speedup vs baseline: 11.4048x; 11.4048x over previous
import functools

import jax
import jax.numpy as jnp
from jax import lax
from jax.experimental import pallas as pl
from jax.experimental.pallas import tpu as pltpu
from jax.experimental.pallas import tpu_sc as plsc

N = 10000
D = 128
E = 320000
L = 16
NC = 2
NS = 16
NW = NC * NS
K = 128
ITERS = 79
EP = NW * K * ITERS
EPW = EP // NW
NP = 10240
ZR = NP // NS
PAD_SRC = N
PAD_DST = N + 8

_mesh = plsc.VectorSubcoreMesh(core_axis_name="c", subcore_axis_name="s")


def _deg_body(dst_hbm, out_hbm, acc, idx_v, ones_v, zrow_v):
    c = lax.axis_index("c")
    s = lax.axis_index("s")
    wid = s * NC + c

    def init(r, _):
        ones_v[r, :] = jnp.ones((L,), jnp.float32)
        zrow_v[r, :] = jnp.zeros((L,), jnp.float32)
        return 0

    lax.fori_loop(0, K, init, 0)

    rbase = s * ZR
    for k in range(ZR // K):
        pltpu.sync_copy(zrow_v, acc.at[pl.ds(rbase + k * K, K)])
    plsc.subcore_barrier()

    ebase = wid * EPW

    def step(g, _):
        pltpu.sync_copy(dst_hbm.at[pl.ds(ebase + g * K, K)], idx_v)
        pltpu.sync_copy(ones_v, acc.at[idx_v], add=True)
        return 0

    lax.fori_loop(0, ITERS, step, 0)
    plsc.subcore_barrier()
    pltpu.sync_copy(acc.at[pl.ds(rbase, ZR)], out_hbm.at[c, pl.ds(rbase, ZR)])


_deg_call = pl.kernel(
    _deg_body,
    out_type=jax.ShapeDtypeStruct((NC, NP, L), jnp.float32),
    mesh=_mesh,
    scratch_types=[
        pltpu.VMEM_SHARED((NP, L), jnp.float32),
        pltpu.VMEM((K,), jnp.int32),
        pltpu.VMEM((K, L), jnp.float32),
        pltpu.VMEM((K, L), jnp.float32),
    ],
)


def _agg_body(h_hbm, src_hbm, dst_hbm, out_hbm, acc, sidx, didx, rows, zrow, gsem):
    c = lax.axis_index("c")
    s = lax.axis_index("s")
    wid = s * NC + c

    def zinit(r, _):
        for j in range(D // L):
            zrow[r, pl.ds(j * L, L)] = jnp.zeros((L,), jnp.float32)
        return 0

    lax.fori_loop(0, K, zinit, 0)

    rbase = s * ZR
    for k in range(ZR // K):
        pltpu.sync_copy(zrow, acc.at[pl.ds(rbase + k * K, K)])
    plsc.subcore_barrier()

    ebase = wid * EPW

    def step(g, _):
        pltpu.sync_copy(src_hbm.at[pl.ds(ebase + g * K, K)], sidx)
        pltpu.sync_copy(dst_hbm.at[pl.ds(ebase + g * K, K)], didx)
        pltpu.async_copy(h_hbm.at[sidx], rows, gsem).wait()
        pltpu.sync_copy(rows, acc.at[didx], add=True)
        return 0

    lax.fori_loop(0, ITERS, step, 0)
    plsc.subcore_barrier()
    pltpu.sync_copy(acc.at[pl.ds(rbase, ZR)], out_hbm.at[c, pl.ds(rbase, ZR)])


_agg_call = pl.kernel(
    _agg_body,
    out_type=jax.ShapeDtypeStruct((NC, NP, D), jnp.float32),
    mesh=_mesh,
    scratch_types=[
        pltpu.VMEM_SHARED((NP, D), jnp.float32),
        pltpu.VMEM((K,), jnp.int32),
        pltpu.VMEM((K,), jnp.int32),
        pltpu.VMEM((K, D), jnp.float32),
        pltpu.VMEM((K, D), jnp.float32),
        pltpu.SemaphoreType.DMA,
    ],
)


_TCR = 1280


def _dinv_of(degp_ref):
    deg = degp_ref[0] + degp_ref[1] + 1.0
    return lax.rsqrt(deg)[:, :1]


def _tc1_body(x_ref, w_ref, degp_ref, out_ref):
    h = jnp.dot(x_ref[...], w_ref[...], preferred_element_type=jnp.float32)
    out_ref[...] = h * _dinv_of(degp_ref)


_tc1_call = pl.pallas_call(
    _tc1_body,
    grid=(NP // _TCR,),
    in_specs=[
        pl.BlockSpec((_TCR, D), lambda i: (i, 0)),
        pl.BlockSpec((D, D), lambda i: (0, 0)),
        pl.BlockSpec((NC, _TCR, L), lambda i: (0, i, 0)),
    ],
    out_specs=pl.BlockSpec((_TCR, D), lambda i: (i, 0)),
    out_shape=jax.ShapeDtypeStruct((NP, D), jnp.float32),
)


def _tc2_body(aggp_ref, h1_ref, degp_ref, b1_ref, w2_ref, out_ref):
    dinv = _dinv_of(degp_ref)
    agg = aggp_ref[0] + aggp_ref[1] + h1_ref[...]
    r = jnp.maximum(agg * dinv + b1_ref[...], 0.0)
    h2 = jnp.dot(r, w2_ref[...], preferred_element_type=jnp.float32) * dinv
    row = pl.program_id(0) * _TCR + lax.broadcasted_iota(jnp.int32, (_TCR, 1), 0)
    out_ref[...] = jnp.where(row < N, h2, 0.0)


_tc2_call = pl.pallas_call(
    _tc2_body,
    grid=(NP // _TCR,),
    in_specs=[
        pl.BlockSpec((NC, _TCR, D), lambda i: (0, i, 0)),
        pl.BlockSpec((_TCR, D), lambda i: (i, 0)),
        pl.BlockSpec((NC, _TCR, L), lambda i: (0, i, 0)),
        pl.BlockSpec((1, D), lambda i: (0, 0)),
        pl.BlockSpec((D, D), lambda i: (0, 0)),
    ],
    out_specs=pl.BlockSpec((_TCR, D), lambda i: (i, 0)),
    out_shape=jax.ShapeDtypeStruct((NP, D), jnp.float32),
)

_TCR3 = 1000


def _tc3_body(aggp_ref, h2_ref, degp_ref, b2_ref, out_ref):
    dinv = _dinv_of(degp_ref)
    out_ref[...] = (aggp_ref[0] + aggp_ref[1] + h2_ref[...]) * dinv + b2_ref[...]


_tc3_call = pl.pallas_call(
    _tc3_body,
    grid=(N // _TCR3,),
    in_specs=[
        pl.BlockSpec((NC, _TCR3, D), lambda i: (0, i, 0)),
        pl.BlockSpec((_TCR3, D), lambda i: (i, 0)),
        pl.BlockSpec((NC, _TCR3, L), lambda i: (0, i, 0)),
        pl.BlockSpec((1, D), lambda i: (0, 0)),
    ],
    out_specs=pl.BlockSpec((_TCR3, D), lambda i: (i, 0)),
    out_shape=jax.ShapeDtypeStruct((N, D), jnp.float32),
)


def kernel(x, edge_index, W1, b1, W2, b2):
    src = edge_index[0].astype(jnp.int32)
    dst = edge_index[1].astype(jnp.int32)
    pad = EP - E
    src_p = jnp.concatenate([src, jnp.full((pad,), PAD_SRC, jnp.int32)])
    dst_p = jnp.concatenate([dst, jnp.full((pad,), PAD_DST, jnp.int32)])
    x_p = jnp.concatenate([x, jnp.zeros((NP - N, D), x.dtype)])

    degp = _deg_call(dst_p)
    h1 = _tc1_call(x_p, W1, degp)
    agg1 = _agg_call(h1, src_p, dst_p)
    h2 = _tc2_call(agg1, h1, degp, b1.reshape(1, D), W2)
    agg2 = _agg_call(h2, src_p, dst_p)
    return _tc3_call(agg2, h2, degp, b2.reshape(1, D))
